# Initial kernel scaffold; baseline (speedup 1.0000x reference)
import jax
import jax.numpy as jnp
from jax.experimental import pallas as pl


def kernel(features, labels, anchor_feats, neg_inds):
    raise NotImplementedError("write your pallas kernel here")


# speedup vs baseline: 17.4807x; 17.4807x over previous
import functools

import jax
import jax.numpy as jnp
from jax.experimental import pallas as pl
from jax.experimental.pallas import tpu as pltpu

N_POINTS = 8192
FEAT_DIM = 512
NUM_LABELS = 200
NUM_NEG = 32
LPAD = 256

POS_THRESH = 0.1
NEG_THRESH = 0.5
EPS = 1e-07

BLK = 1024
GRID = N_POINTS // BLK


def _loss_body(x_ref, at_ref, lab_ref, neg_ref, pos_ref, neg_out_ref):
    f = x_ref[...]
    at = at_ref[...]
    g = jnp.dot(f, at, preferred_element_type=jnp.float32)
    fn = jnp.sum(f * f, axis=1, keepdims=True)
    an = jnp.sum(at * at, axis=0, keepdims=True)
    d2 = jnp.maximum(fn + an - 2.0 * g, 0.0)
    s = jnp.sqrt(d2 + EPS)

    cols = jax.lax.broadcasted_iota(jnp.int32, (BLK, LPAD), 1)
    lab = lab_ref[...]
    pos_d = jnp.sum(jnp.where(cols == lab, s, 0.0), axis=1, keepdims=True)

    acc = jnp.zeros((BLK, LPAD), dtype=jnp.float32)
    for k in range(NUM_NEG):
        nk = neg_ref[:, k][:, None]
        acc = acc + jnp.where(cols == nk, s, 0.0)
    neg_d = jnp.sum(acc, axis=1, keepdims=True) / NUM_NEG

    pos_ref[...] = jnp.maximum(pos_d - POS_THRESH, 0.0)
    neg_out_ref[...] = jnp.maximum(NEG_THRESH - neg_d, 0.0)


@jax.jit
def _run(features, labels, anchor_feats, neg_inds):
    at = jnp.zeros((FEAT_DIM, LPAD), jnp.float32).at[:, :NUM_LABELS].set(
        anchor_feats.T)
    lab2 = labels[:, None]
    pos, neg = pl.pallas_call(
        _loss_body,
        grid=(GRID,),
        in_specs=[
            pl.BlockSpec((BLK, FEAT_DIM), lambda i: (i, 0)),
            pl.BlockSpec((FEAT_DIM, LPAD), lambda i: (0, 0)),
            pl.BlockSpec((BLK, 1), lambda i: (i, 0)),
            pl.BlockSpec((BLK, NUM_NEG), lambda i: (i, 0)),
        ],
        out_specs=[
            pl.BlockSpec((BLK, 1), lambda i: (i, 0)),
            pl.BlockSpec((BLK, 1), lambda i: (i, 0)),
        ],
        out_shape=[
            jax.ShapeDtypeStruct((N_POINTS, 1), jnp.float32),
            jax.ShapeDtypeStruct((N_POINTS, 1), jnp.float32),
        ],
    )(features, at, lab2, neg_inds)
    pos = pos[:, 0]
    neg = neg[:, 0]
    loss = pos.mean() + neg.mean()
    return (loss, pos, neg)


def kernel(features, labels, anchor_feats, neg_inds):
    return _run(features, labels, anchor_feats, neg_inds)


# speedup vs baseline: 19.7212x; 1.1282x over previous
import functools

import jax
import jax.numpy as jnp
from jax import lax
from jax.experimental import pallas as pl
from jax.experimental.pallas import tpu as pltpu
from jax.experimental.pallas import tpu_sc as plsc

N_POINTS = 8192
FEAT_DIM = 512
NUM_LABELS = 200
NUM_NEG = 32

POS_THRESH = 0.1
NEG_THRESH = 0.5
EPS = 1e-07

BLK = 1024
GRID = N_POINTS // BLK

NW = 32
ROWS = N_POINTS // NW
GROUPS = ROWS // 16


def _dist_body(x_ref, a_ref, s_ref):
    f = x_ref[...]
    a = a_ref[...]
    g = lax.dot_general(f, a, (((1,), (1,)), ((), ())),
                        preferred_element_type=jnp.float32)
    fn = jnp.sum(f * f, axis=1, keepdims=True)
    ones = jnp.ones((1, FEAT_DIM), jnp.float32)
    an = lax.dot_general(ones, a * a, (((1,), (1,)), ((), ())),
                         preferred_element_type=jnp.float32)
    d2 = jnp.maximum(fn + an - 2.0 * g, 0.0)
    s_ref[...] = jnp.sqrt(d2 + EPS)


_sc_mesh = plsc.VectorSubcoreMesh(core_axis_name="c", subcore_axis_name="s")


@functools.partial(
    pl.kernel,
    mesh=_sc_mesh,
    compiler_params=pltpu.CompilerParams(
        use_tc_tiling_on_sc=False, needs_layout_passes=False),
    out_type=[
        jax.ShapeDtypeStruct((N_POINTS,), jnp.float32),
        jax.ShapeDtypeStruct((N_POINTS,), jnp.float32),
    ],
    scratch_types=[
        pltpu.VMEM((ROWS, NUM_LABELS), jnp.float32),
        pltpu.VMEM((ROWS,), jnp.int32),
        pltpu.VMEM((ROWS, NUM_NEG), jnp.int32),
        pltpu.VMEM((ROWS,), jnp.float32),
        pltpu.VMEM((ROWS,), jnp.float32),
    ],
)
def _sc_gather(s_hbm, lab_hbm, neg_hbm, pos_hbm, negout_hbm,
               s_v, lab_v, neg_v, pos_v, nout_v):
    wid = lax.axis_index("s") * 2 + lax.axis_index("c")
    base = wid * ROWS
    pltpu.sync_copy(s_hbm.at[pl.ds(base, ROWS)], s_v)
    pltpu.sync_copy(lab_hbm.at[pl.ds(base, ROWS)], lab_v)
    pltpu.sync_copy(neg_hbm.at[pl.ds(base, ROWS)], neg_v)

    def group(g, _):
        rows = lax.iota(jnp.int32, 16) + g * 16
        lab = lab_v[pl.ds(g * 16, 16)]
        dpos = plsc.load_gather(s_v, [rows, lab])
        pos_v[pl.ds(g * 16, 16)] = jnp.maximum(dpos - POS_THRESH, 0.0)
        acc = jnp.zeros((16,), jnp.float32)
        for k in range(NUM_NEG):
            nk = plsc.load_gather(neg_v, [rows, jnp.full((16,), k, jnp.int32)])
            acc = acc + plsc.load_gather(s_v, [rows, nk])
        nout_v[pl.ds(g * 16, 16)] = jnp.maximum(
            NEG_THRESH - acc * (1.0 / NUM_NEG), 0.0)
        return 0

    lax.fori_loop(0, GROUPS, group, 0)
    pltpu.sync_copy(pos_v, pos_hbm.at[pl.ds(base, ROWS)])
    pltpu.sync_copy(nout_v, negout_hbm.at[pl.ds(base, ROWS)])


@jax.jit
def _run(features, labels, anchor_feats, neg_inds):
    sqrt_d = pl.pallas_call(
        _dist_body,
        grid=(GRID,),
        in_specs=[
            pl.BlockSpec((BLK, FEAT_DIM), lambda i: (i, 0)),
            pl.BlockSpec((NUM_LABELS, FEAT_DIM), lambda i: (0, 0)),
        ],
        out_specs=pl.BlockSpec((BLK, NUM_LABELS), lambda i: (i, 0)),
        out_shape=jax.ShapeDtypeStruct((N_POINTS, NUM_LABELS), jnp.float32),
    )(features, anchor_feats)
    pos, neg = _sc_gather(sqrt_d, labels, neg_inds)
    loss = pos.mean() + neg.mean()
    return (loss, pos, neg)


def kernel(features, labels, anchor_feats, neg_inds):
    return _run(features, labels, anchor_feats, neg_inds)


# speedup vs baseline: 25.8987x; 1.3132x over previous
import functools

import jax
import jax.numpy as jnp
from jax import lax
from jax.experimental import pallas as pl
from jax.experimental.pallas import tpu as pltpu
from jax.experimental.pallas import tpu_sc as plsc

N_POINTS = 8192
FEAT_DIM = 512
NUM_LABELS = 200
NUM_NEG = 32
LPAD = 256

POS_THRESH = 0.1
NEG_THRESH = 0.5
EPS = 1e-07

BLK = 1024
GRID = N_POINTS // BLK

NW = 32
ROWS = N_POINTS // NW
GROUPS = ROWS // 16


def _dist_body(x_ref, a_ref, sl_ref, sr_ref):
    f = x_ref[...]
    a = a_ref[...]
    g = lax.dot_general(f, a, (((1,), (1,)), ((), ())),
                        preferred_element_type=jnp.float32)
    fn = jnp.sum(f * f, axis=1, keepdims=True)
    ones = jnp.ones((1, FEAT_DIM), jnp.float32)
    an = lax.dot_general(ones, a * a, (((1,), (1,)), ((), ())),
                         preferred_element_type=jnp.float32)
    d2 = jnp.maximum(fn + an - 2.0 * g, 0.0)
    s = jnp.sqrt(d2 + EPS)
    sl_ref[...] = s[:, :128]
    sr_ref[...] = s[:, 128:]


_sc_mesh = plsc.VectorSubcoreMesh(core_axis_name="c", subcore_axis_name="s")


@functools.partial(
    pl.kernel,
    mesh=_sc_mesh,
    compiler_params=pltpu.CompilerParams(
        use_tc_tiling_on_sc=False, needs_layout_passes=False),
    out_type=[
        jax.ShapeDtypeStruct((N_POINTS,), jnp.float32),
        jax.ShapeDtypeStruct((N_POINTS,), jnp.float32),
        jax.ShapeDtypeStruct((NW, 32), jnp.float32),
    ],
    scratch_types=[
        pltpu.VMEM((2, ROWS, 128), jnp.float32),
        pltpu.VMEM((ROWS,), jnp.int32),
        pltpu.VMEM((NUM_NEG, ROWS), jnp.int32),
        pltpu.VMEM((ROWS,), jnp.float32),
        pltpu.VMEM((ROWS,), jnp.float32),
        pltpu.VMEM((32,), jnp.float32),
    ],
)
def _sc_gather(sl_hbm, sr_hbm, lab_hbm, negt_hbm, pos_hbm, negout_hbm,
               part_hbm, s_v, lab_v, neg_v, pos_v, nout_v, psum_v):
    wid = lax.axis_index("s") * 2 + lax.axis_index("c")
    base = wid * ROWS
    pltpu.sync_copy(sl_hbm.at[pl.ds(base, ROWS)], s_v.at[0])
    pltpu.sync_copy(sr_hbm.at[pl.ds(base, ROWS)], s_v.at[1])
    pltpu.sync_copy(lab_hbm.at[pl.ds(base, ROWS)], lab_v)
    pltpu.sync_copy(negt_hbm.at[:, pl.ds(base, ROWS)], neg_v)

    def group(g, carry):
        pos_acc, neg_acc = carry
        rows = lax.iota(jnp.int32, 16) + g * 16
        lab = lab_v[pl.ds(g * 16, 16)]
        dpos = plsc.load_gather(
            s_v, [lab >> 7, rows, lab & 127])
        pos_val = jnp.maximum(dpos - POS_THRESH, 0.0)
        pos_v[pl.ds(g * 16, 16)] = pos_val
        acc = jnp.zeros((16,), jnp.float32)
        for k in range(NUM_NEG):
            nk = neg_v[k, pl.ds(g * 16, 16)]
            acc = acc + plsc.load_gather(
                s_v, [nk >> 7, rows, nk & 127])
        neg_val = jnp.maximum(NEG_THRESH - acc * (1.0 / NUM_NEG), 0.0)
        nout_v[pl.ds(g * 16, 16)] = neg_val
        return pos_acc + pos_val, neg_acc + neg_val

    zeros16 = jnp.zeros((16,), jnp.float32)
    pos_acc, neg_acc = lax.fori_loop(0, GROUPS, group, (zeros16, zeros16))
    psum_v[pl.ds(0, 16)] = pos_acc
    psum_v[pl.ds(16, 16)] = neg_acc
    pltpu.sync_copy(pos_v, pos_hbm.at[pl.ds(base, ROWS)])
    pltpu.sync_copy(nout_v, negout_hbm.at[pl.ds(base, ROWS)])
    pltpu.sync_copy(psum_v, part_hbm.at[wid])


@jax.jit
def _run(features, labels, anchor_feats, neg_inds):
    a_pad = jnp.zeros((LPAD, FEAT_DIM), jnp.float32).at[:NUM_LABELS].set(
        anchor_feats)
    sl, sr = pl.pallas_call(
        _dist_body,
        grid=(GRID,),
        in_specs=[
            pl.BlockSpec((BLK, FEAT_DIM), lambda i: (i, 0)),
            pl.BlockSpec((LPAD, FEAT_DIM), lambda i: (0, 0)),
        ],
        out_specs=[
            pl.BlockSpec((BLK, 128), lambda i: (i, 0)),
            pl.BlockSpec((BLK, 128), lambda i: (i, 0)),
        ],
        out_shape=[
            jax.ShapeDtypeStruct((N_POINTS, 128), jnp.float32),
            jax.ShapeDtypeStruct((N_POINTS, 128), jnp.float32),
        ],
    )(features, a_pad)
    pos, neg, part = _sc_gather(sl, sr, labels, neg_inds.T)
    loss = part.sum() * (1.0 / N_POINTS)
    return (loss, pos, neg)


def kernel(features, labels, anchor_feats, neg_inds):
    return _run(features, labels, anchor_feats, neg_inds)
